# baseline (speedup 1.0000x reference)
import functools

import jax
import jax.numpy as jnp
import numpy as np
from jax import lax
from jax.experimental import pallas as pl
from jax.experimental.pallas import tpu as pltpu
from jax.experimental.pallas import tpu_sc as plsc

_NUM_SC_CORES = 2
_NUM_SC_SUBCORES = 16
_NUM_WORKERS = _NUM_SC_CORES * _NUM_SC_SUBCORES


def _pe_table(seq_len, d_model):
    pe = np.zeros((seq_len, d_model), dtype=np.float32)
    position = np.arange(0, seq_len).astype(np.float32)[:, None]
    div_term = np.exp(
        np.arange(0, d_model, 2).astype(np.float32) * -(np.log(10000.0) / d_model)
    )
    pe[:, 0::2] = np.sin(position * div_term)
    pe[:, 1::2] = np.cos(position * div_term)
    return pe


def _sc_gather(emb_table, idx_flat):
    n = idx_flat.shape[0]
    d = emb_table.shape[1]
    b_per_w = n // _NUM_WORKERS
    mesh = plsc.VectorSubcoreMesh(core_axis_name="c", subcore_axis_name="s")

    chunk = 400
    n_chunks = b_per_w // chunk

    @functools.partial(
        pl.kernel,
        mesh=mesh,
        out_type=jax.ShapeDtypeStruct((n, d), jnp.float32),
        scratch_types=[
            pltpu.VMEM((b_per_w,), jnp.int32),
            pltpu.VMEM((2, chunk, d), jnp.float32),
            pltpu.SemaphoreType.DMA,
            pltpu.SemaphoreType.DMA,
        ],
    )
    def gather_kernel(table_hbm, idx_hbm, out_hbm, idx_v, rows_v, sem0, sem1):
        sems = [sem0, sem1]
        wid = lax.axis_index("s") * _NUM_SC_CORES + lax.axis_index("c")
        base = wid * b_per_w
        pltpu.sync_copy(idx_hbm.at[pl.ds(base, b_per_w)], idx_v)
        copies = [None, None]
        copies[0] = pltpu.async_copy(
            table_hbm.at[idx_v.at[pl.ds(0, chunk)]], rows_v.at[0], sems[0]
        )
        for c in range(n_chunks):
            cur, nxt = c % 2, (c + 1) % 2
            copies[cur].wait()
            if c + 1 < n_chunks:
                copies[nxt] = pltpu.async_copy(
                    table_hbm.at[idx_v.at[pl.ds((c + 1) * chunk, chunk)]],
                    rows_v.at[nxt],
                    sems[nxt],
                )
            pltpu.sync_copy(
                rows_v.at[cur], out_hbm.at[pl.ds(base + c * chunk, chunk)]
            )

    return gather_kernel(emb_table, idx_flat)


def _tc_block_body(h_ref, pe_ref, w_ref, b_ref, o_ref, *, d_model):
    z = jnp.tanh(h_ref[:, :d_model] + pe_ref[...])
    o_ref[...] = (
        jnp.dot(z, w_ref[...], preferred_element_type=jnp.float32) + b_ref[...]
    )


def kernel(x, emb_table, W, b):
    batch, seq_len = x.shape
    vocab, d_model = emb_table.shape
    n_tok = batch * seq_len

    d_pad = 128
    emb_padded = jnp.pad(emb_table, ((0, 0), (0, d_pad - d_model)))
    idx_flat = x.reshape(n_tok).astype(jnp.int32)
    hidden = _sc_gather(emb_padded, idx_flat)

    seqs_per_block = 32
    blk = seqs_per_block * seq_len
    n_blocks = n_tok // blk

    pe = _pe_table(seq_len, d_model)
    pe_tiled = jnp.asarray(np.tile(pe, (seqs_per_block, 1)))
    b2 = b.reshape(1, vocab)

    logits = pl.pallas_call(
        functools.partial(_tc_block_body, d_model=d_model),
        grid=(n_blocks,),
        in_specs=[
            pl.BlockSpec((blk, d_pad), lambda i: (i, 0)),
            pl.BlockSpec((blk, d_model), lambda i: (0, 0)),
            pl.BlockSpec((d_model, vocab), lambda i: (0, 0)),
            pl.BlockSpec((1, vocab), lambda i: (0, 0)),
        ],
        out_specs=pl.BlockSpec((blk, vocab), lambda i: (i, 0)),
        out_shape=jax.ShapeDtypeStruct((n_tok, vocab), jnp.float32),
        compiler_params=pltpu.CompilerParams(
            dimension_semantics=("parallel",),
        ),
    )(hidden, pe_tiled, W, b2)

    return logits.reshape(batch, seq_len, vocab)


# speedup vs baseline: 3.2470x; 3.2470x over previous
import functools

import jax
import jax.numpy as jnp
import numpy as np
from jax import lax
from jax.experimental import pallas as pl
from jax.experimental.pallas import tpu as pltpu
from jax.experimental.pallas import tpu_sc as plsc

_NUM_SC_CORES = 2
_NUM_SC_SUBCORES = 16
_NUM_WORKERS = _NUM_SC_CORES * _NUM_SC_SUBCORES


def _pe_table(seq_len, d_model):
    pe = np.zeros((seq_len, d_model), dtype=np.float32)
    position = np.arange(0, seq_len).astype(np.float32)[:, None]
    div_term = np.exp(
        np.arange(0, d_model, 2).astype(np.float32) * -(np.log(10000.0) / d_model)
    )
    pe[:, 0::2] = np.sin(position * div_term)
    pe[:, 1::2] = np.cos(position * div_term)
    return pe


def _sc_gather(emb_padded, idx_flat):
    n = idx_flat.shape[0]
    d = emb_padded.shape[1]
    b_per_w = n // _NUM_WORKERS
    mesh = plsc.VectorSubcoreMesh(core_axis_name="c", subcore_axis_name="s")

    chunk = 400
    n_chunks = b_per_w // chunk

    @functools.partial(
        pl.kernel,
        mesh=mesh,
        out_type=jax.ShapeDtypeStruct((n, d), jnp.float32),
        scratch_types=[
            pltpu.VMEM((b_per_w,), jnp.int32),
            pltpu.VMEM((2, chunk, d), jnp.float32),
            pltpu.SemaphoreType.DMA,
            pltpu.SemaphoreType.DMA,
        ],
    )
    def gather_kernel(table_hbm, idx_hbm, out_hbm, idx_v, rows_v, sem0, sem1):
        sems = [sem0, sem1]
        wid = lax.axis_index("s") * _NUM_SC_CORES + lax.axis_index("c")
        base = wid * b_per_w
        pltpu.sync_copy(idx_hbm.at[pl.ds(base, b_per_w)], idx_v)
        copies = [None, None]
        copies[0] = pltpu.async_copy(
            table_hbm.at[idx_v.at[pl.ds(0, chunk)]], rows_v.at[0], sems[0]
        )
        for c in range(n_chunks):
            cur, nxt = c % 2, (c + 1) % 2
            copies[cur].wait()
            if c + 1 < n_chunks:
                copies[nxt] = pltpu.async_copy(
                    table_hbm.at[idx_v.at[pl.ds((c + 1) * chunk, chunk)]],
                    rows_v.at[nxt],
                    sems[nxt],
                )
            pltpu.sync_copy(
                rows_v.at[cur], out_hbm.at[pl.ds(base + c * chunk, chunk)]
            )

    return gather_kernel(emb_padded, idx_flat)


def _tc_block_body(h_ref, pe_ref, wt_ref, b_ref, o_ref, *, d_model):
    z = jnp.tanh(h_ref[:, :d_model] + pe_ref[0])
    o = lax.dot_general(
        wt_ref[...],
        z,
        (((1,), (1,)), ((), ())),
        preferred_element_type=jnp.float32,
    )
    o_ref[0] = o + b_ref[...]


def kernel(x, emb_table, W, b):
    batch, seq_len = x.shape
    vocab, d_model = emb_table.shape
    n_tok = batch * seq_len

    d_pad = 128
    emb_padded = jnp.pad(emb_table, ((0, 0), (0, d_pad - d_model)))
    idx_flat = x.T.reshape(n_tok).astype(jnp.int32)
    hidden = _sc_gather(emb_padded, idx_flat)

    pe3 = jnp.asarray(_pe_table(seq_len, d_model)).reshape(seq_len, 1, d_model)
    w_t = W.T
    b_col = b.reshape(vocab, 1)

    out_phys = pl.pallas_call(
        functools.partial(_tc_block_body, d_model=d_model),
        grid=(seq_len,),
        in_specs=[
            pl.BlockSpec((batch, d_pad), lambda s: (s, 0)),
            pl.BlockSpec((1, 1, d_model), lambda s: (s, 0, 0)),
            pl.BlockSpec((vocab, d_model), lambda s: (0, 0)),
            pl.BlockSpec((vocab, 1), lambda s: (0, 0)),
        ],
        out_specs=pl.BlockSpec((1, vocab, batch), lambda s: (s, 0, 0)),
        out_shape=jax.ShapeDtypeStruct((seq_len, vocab, batch), jnp.float32),
        compiler_params=pltpu.CompilerParams(
            dimension_semantics=("parallel",),
        ),
    )(hidden, pe3, w_t, b_col)

    return out_phys.transpose(2, 0, 1)


# speedup vs baseline: 3.5701x; 1.0995x over previous
import functools

import jax
import jax.numpy as jnp
import numpy as np
from jax import lax
from jax.experimental import pallas as pl
from jax.experimental.pallas import tpu as pltpu
from jax.experimental.pallas import tpu_sc as plsc

_NUM_SC_CORES = 2
_NUM_SC_SUBCORES = 16
_NUM_WORKERS = _NUM_SC_CORES * _NUM_SC_SUBCORES


def _pe_table(seq_len, d_model):
    pe = np.zeros((seq_len, d_model), dtype=np.float32)
    position = np.arange(0, seq_len).astype(np.float32)[:, None]
    div_term = np.exp(
        np.arange(0, d_model, 2).astype(np.float32) * -(np.log(10000.0) / d_model)
    )
    pe[:, 0::2] = np.sin(position * div_term)
    pe[:, 1::2] = np.cos(position * div_term)
    return pe


def _sc_gather(emb_padded, idx_flat):
    n = idx_flat.shape[0]
    d = emb_padded.shape[1]
    b_per_w = n // _NUM_WORKERS
    mesh = plsc.VectorSubcoreMesh(core_axis_name="c", subcore_axis_name="s")

    chunk = 400
    n_chunks = b_per_w // chunk

    @functools.partial(
        pl.kernel,
        mesh=mesh,
        out_type=jax.ShapeDtypeStruct((n, d), jnp.float32),
        scratch_types=[
            pltpu.VMEM((b_per_w,), jnp.int32),
            pltpu.VMEM((2, chunk, d), jnp.float32),
            pltpu.SemaphoreType.DMA,
            pltpu.SemaphoreType.DMA,
        ],
    )
    def gather_kernel(table_hbm, idx_hbm, out_hbm, idx_v, rows_v, sem0, sem1):
        sems = [sem0, sem1]
        wid = lax.axis_index("s") * _NUM_SC_CORES + lax.axis_index("c")
        base = wid * b_per_w
        pltpu.sync_copy(idx_hbm.at[pl.ds(base, b_per_w)], idx_v)
        copies = [None, None]
        copies[0] = pltpu.async_copy(
            table_hbm.at[idx_v.at[pl.ds(0, chunk)]], rows_v.at[0], sems[0]
        )
        for c in range(n_chunks):
            cur, nxt = c % 2, (c + 1) % 2
            copies[cur].wait()
            if c + 1 < n_chunks:
                copies[nxt] = pltpu.async_copy(
                    table_hbm.at[idx_v.at[pl.ds((c + 1) * chunk, chunk)]],
                    rows_v.at[nxt],
                    sems[nxt],
                )
            pltpu.sync_copy(
                rows_v.at[cur], out_hbm.at[pl.ds(base + c * chunk, chunk)]
            )

    return gather_kernel(emb_padded, idx_flat)


def _tc_block_body(h_ref, pe_ref, wt_ref, b_ref, o_ref, *, d_model, s_per_blk, batch):
    for j in range(s_per_blk):
        z = jnp.tanh(
            h_ref[pl.ds(j * batch, batch), :d_model] + pe_ref[j]
        )
        o = lax.dot_general(
            wt_ref[...],
            z,
            (((1,), (1,)), ((), ())),
            preferred_element_type=jnp.float32,
        )
        o_ref[j] = o + b_ref[...]


def kernel(x, emb_table, W, b):
    batch, seq_len = x.shape
    vocab, d_model = emb_table.shape
    n_tok = batch * seq_len

    d_pad = 128
    emb_padded = jnp.pad(emb_table, ((0, 0), (0, d_pad - d_model)))
    idx_flat = x.T.reshape(n_tok).astype(jnp.int32)
    hidden = _sc_gather(emb_padded, idx_flat)

    pe3 = jnp.asarray(_pe_table(seq_len, d_model)).reshape(seq_len, 1, d_model)
    w_t = W.T
    b_col = b.reshape(vocab, 1)

    s_per_blk = 2
    out_phys = pl.pallas_call(
        functools.partial(
            _tc_block_body, d_model=d_model, s_per_blk=s_per_blk, batch=batch
        ),
        grid=(seq_len // s_per_blk,),
        in_specs=[
            pl.BlockSpec((s_per_blk * batch, d_pad), lambda s: (s, 0)),
            pl.BlockSpec((s_per_blk, 1, d_model), lambda s: (s, 0, 0)),
            pl.BlockSpec((vocab, d_model), lambda s: (0, 0)),
            pl.BlockSpec((vocab, 1), lambda s: (0, 0)),
        ],
        out_specs=pl.BlockSpec(
            (s_per_blk, vocab, batch), lambda s: (s, 0, 0)
        ),
        out_shape=jax.ShapeDtypeStruct((seq_len, vocab, batch), jnp.float32),
        compiler_params=pltpu.CompilerParams(
            dimension_semantics=("parallel",),
        ),
    )(hidden, pe3, w_t, b_col)

    return out_phys.transpose(2, 0, 1)


# speedup vs baseline: 3.5726x; 1.0007x over previous
import functools

import jax
import jax.numpy as jnp
import numpy as np
from jax import lax
from jax.experimental import pallas as pl
from jax.experimental.pallas import tpu as pltpu
from jax.experimental.pallas import tpu_sc as plsc

_NUM_SC_CORES = 2
_NUM_SC_SUBCORES = 16
_NUM_WORKERS = _NUM_SC_CORES * _NUM_SC_SUBCORES


def _pe_table(seq_len, d_model):
    pe = np.zeros((seq_len, d_model), dtype=np.float32)
    position = np.arange(0, seq_len).astype(np.float32)[:, None]
    div_term = np.exp(
        np.arange(0, d_model, 2).astype(np.float32) * -(np.log(10000.0) / d_model)
    )
    pe[:, 0::2] = np.sin(position * div_term)
    pe[:, 1::2] = np.cos(position * div_term)
    return pe


def _sc_gather(emb_padded, idx_flat):
    n = idx_flat.shape[0]
    d = emb_padded.shape[1]
    b_per_w = n // _NUM_WORKERS
    mesh = plsc.VectorSubcoreMesh(core_axis_name="c", subcore_axis_name="s")

    chunk = 400
    n_chunks = b_per_w // chunk

    @functools.partial(
        pl.kernel,
        mesh=mesh,
        out_type=jax.ShapeDtypeStruct((n, d), jnp.float32),
        scratch_types=[
            pltpu.VMEM((b_per_w,), jnp.int32),
            pltpu.VMEM((2, chunk, d), jnp.float32),
            pltpu.SemaphoreType.DMA,
            pltpu.SemaphoreType.DMA,
        ],
    )
    def gather_kernel(table_hbm, idx_hbm, out_hbm, idx_v, rows_v, sem0, sem1):
        sems = [sem0, sem1]
        wid = lax.axis_index("s") * _NUM_SC_CORES + lax.axis_index("c")
        base = wid * b_per_w
        pltpu.sync_copy(idx_hbm.at[pl.ds(base, b_per_w)], idx_v)
        copies = [None, None]
        copies[0] = pltpu.async_copy(
            table_hbm.at[idx_v.at[pl.ds(0, chunk)]], rows_v.at[0], sems[0]
        )
        for c in range(n_chunks):
            cur, nxt = c % 2, (c + 1) % 2
            copies[cur].wait()
            if c + 1 < n_chunks:
                copies[nxt] = pltpu.async_copy(
                    table_hbm.at[idx_v.at[pl.ds((c + 1) * chunk, chunk)]],
                    rows_v.at[nxt],
                    sems[nxt],
                )
            pltpu.sync_copy(
                rows_v.at[cur], out_hbm.at[pl.ds(base + c * chunk, chunk)]
            )

    return gather_kernel(emb_padded, idx_flat)


def _tc_block_body(h_ref, pe_ref, wt_ref, b_ref, o_ref, *, d_model, s_per_blk, batch):
    for j in range(s_per_blk):
        z = jnp.tanh(
            h_ref[pl.ds(j * batch, batch), :d_model] + pe_ref[j]
        )
        o = lax.dot_general(
            wt_ref[...],
            z,
            (((1,), (1,)), ((), ())),
            preferred_element_type=jnp.float32,
        )
        o_ref[j] = o + b_ref[...]


def kernel(x, emb_table, W, b):
    batch, seq_len = x.shape
    vocab, d_model = emb_table.shape
    n_tok = batch * seq_len

    d_pad = 128
    emb_padded = jnp.pad(emb_table, ((0, 0), (0, d_pad - d_model)))
    idx_flat = x.T.reshape(n_tok).astype(jnp.int32)
    hidden = _sc_gather(emb_padded, idx_flat)

    pe3 = jnp.asarray(_pe_table(seq_len, d_model)).reshape(seq_len, 1, d_model)
    w_t = W.T
    b_col = b.reshape(vocab, 1)

    s_per_blk = 5
    out_phys = pl.pallas_call(
        functools.partial(
            _tc_block_body, d_model=d_model, s_per_blk=s_per_blk, batch=batch
        ),
        grid=(seq_len // s_per_blk,),
        in_specs=[
            pl.BlockSpec((s_per_blk * batch, d_pad), lambda s: (s, 0)),
            pl.BlockSpec((s_per_blk, 1, d_model), lambda s: (s, 0, 0)),
            pl.BlockSpec((vocab, d_model), lambda s: (0, 0)),
            pl.BlockSpec((vocab, 1), lambda s: (0, 0)),
        ],
        out_specs=pl.BlockSpec(
            (s_per_blk, vocab, batch), lambda s: (s, 0, 0)
        ),
        out_shape=jax.ShapeDtypeStruct((seq_len, vocab, batch), jnp.float32),
        compiler_params=pltpu.CompilerParams(
            dimension_semantics=("parallel",),
        ),
    )(hidden, pe3, w_t, b_col)

    return out_phys.transpose(2, 0, 1)
